# baseline (speedup 1.0000x reference)
import functools

import jax
import jax.numpy as jnp
from jax import lax
from jax.experimental import pallas as pl
from jax.experimental.pallas import tpu as pltpu
from jax.experimental.pallas import tpu_sc as plsc

N = 10000
E = 320000
D = 128
NG = 256
NP = 10240
CH = 128
NCH0 = 56
NCH1 = 104
EPAD = 16 * (NCH0 + NCH1) * CH
NPOOL = 512
PCH = 8
PCW = 40
EPOOL = 32 * PCH * PCW
BR = 400
G = N // BR

_MESH = plsc.VectorSubcoreMesh(core_axis_name="c", subcore_axis_name="s")


def _make_sc_scatter(n_rows, n0, n1, ch):
    rp = n_rows // 16
    nmax = max(n0, n1)

    @functools.partial(
        pl.kernel,
        out_type=jax.ShapeDtypeStruct((2, n_rows, 128), jnp.float32),
        mesh=_MESH,
        scratch_types=[
            pltpu.VMEM((nmax, ch), jnp.int32),
            pltpu.VMEM((nmax, ch), jnp.int32),
            pltpu.VMEM((ch, 128), jnp.float32),
            pltpu.VMEM_SHARED((n_rows, 128), jnp.float32),
            pltpu.SemaphoreType.DMA,
        ],
        compiler_params=pltpu.CompilerParams(needs_layout_passes=False),
    )
    def k(y_hbm, src_hbm, dst_hbm, zeros_hbm, out_hbm, src_v, dst_v, rows_v,
          acc_sh, sem):
        c = lax.axis_index("c")
        s = lax.axis_index("s")
        pltpu.sync_copy(zeros_hbm.at[pl.ds(0, rp)], acc_sh.at[pl.ds(s * rp, rp)])

        def run(nc, base):
            if nc == nmax:
                pltpu.sync_copy(src_hbm.at[pl.ds(base, nc)], src_v)
                pltpu.sync_copy(dst_hbm.at[pl.ds(base, nc)], dst_v)
            else:
                pltpu.sync_copy(src_hbm.at[pl.ds(base, nc)],
                                src_v.at[pl.ds(0, nc)])
                pltpu.sync_copy(dst_hbm.at[pl.ds(base, nc)],
                                dst_v.at[pl.ds(0, nc)])

            def body(j, carry):
                pltpu.async_copy(y_hbm.at[src_v.at[j]], rows_v, sem).wait()
                pltpu.sync_copy(rows_v, acc_sh.at[dst_v.at[j]], add=True)
                return carry

            lax.fori_loop(0, nc, body, 0)

        if n0 == n1:
            run(n0, (c * 16 + s) * n0)
        else:
            @pl.when(c == 0)
            def _():
                run(n0, s * n0)

            @pl.when(c == 1)
            def _():
                run(n1, 16 * n0 + s * n1)

        plsc.subcore_barrier()
        pltpu.sync_copy(acc_sh.at[pl.ds(s * rp, rp)],
                        out_hbm.at[c].at[pl.ds(s * rp, rp)])

    return k


def _sc_hists(dst, bat_pad):
    pt_d = E // 32
    pt_b = NP // 32

    @functools.partial(
        pl.kernel,
        out_type=[jax.ShapeDtypeStruct((32, NP), jnp.float32),
                  jax.ShapeDtypeStruct((32, NPOOL), jnp.float32)],
        mesh=_MESH,
        scratch_types=[
            pltpu.VMEM((pt_d,), jnp.int32),
            pltpu.VMEM((pt_b,), jnp.int32),
            pltpu.VMEM((NP,), jnp.float32),
            pltpu.VMEM((NPOOL,), jnp.float32),
        ],
        compiler_params=pltpu.CompilerParams(needs_layout_passes=False),
    )
    def k(dst_hbm, bat_hbm, deg_hbm, cnt_hbm, dv, bv, dh, chh):
        c = lax.axis_index("c")
        s = lax.axis_index("s")
        wid = c * 16 + s
        pltpu.sync_copy(dst_hbm.at[pl.ds(wid * pt_d, pt_d)], dv)
        pltpu.sync_copy(bat_hbm.at[pl.ds(wid * pt_b, pt_b)], bv)
        zeros = jnp.zeros((16,), jnp.float32)
        ones = jnp.ones((16,), jnp.float32)

        def zd(j, carry):
            dh[pl.ds(j * 16, 16)] = zeros
            return carry

        lax.fori_loop(0, NP // 16, zd, 0)

        def zc(j, carry):
            chh[pl.ds(j * 16, 16)] = zeros
            return carry

        lax.fori_loop(0, NPOOL // 16, zc, 0)

        def bd(j, carry):
            plsc.addupdate_scatter(dh, [dv[pl.ds(j * 16, 16)]], ones)
            return carry

        lax.fori_loop(0, pt_d // 16, bd, 0)

        def bb(j, carry):
            plsc.addupdate_scatter(chh, [bv[pl.ds(j * 16, 16)]], ones)
            return carry

        lax.fori_loop(0, pt_b // 16, bb, 0)
        pltpu.sync_copy(dh, deg_hbm.at[wid])
        pltpu.sync_copy(chh, cnt_hbm.at[wid])

    return k(dst, bat_pad)


_sc_scatter_edges = _make_sc_scatter(NP, NCH0, NCH1, CH)
_sc_scatter_pool = _make_sc_scatter(NPOOL, PCH, PCH, PCW)


def _tc_dinv(histT):
    def body(h_ref, d_ref):
        deg = jnp.sum(h_ref[...], axis=1, keepdims=True) + 1.0
        d_ref[...] = lax.rsqrt(deg)

    return pl.pallas_call(
        body,
        grid=(G,),
        in_specs=[pl.BlockSpec((BR, 32), lambda i: (i, 0))],
        out_specs=pl.BlockSpec((BR, 1), lambda i: (i, 0)),
        out_shape=jax.ShapeDtypeStruct((N, 1), jnp.float32),
    )(histT)


def _tc_proj_matmul(x, posp, Wx, Wpp, b, W0, dinv):
    def body(x_ref, p_ref, wx_ref, wp_ref, b_ref, w0_ref, d_ref, y_ref):
        h = jnp.maximum(
            jnp.dot(x_ref[...], wx_ref[...], preferred_element_type=jnp.float32)
            + jnp.dot(p_ref[...], wp_ref[...], preferred_element_type=jnp.float32)
            + b_ref[...], 0.0)
        y_ref[...] = d_ref[...] * jnp.dot(
            h, w0_ref[...], preferred_element_type=jnp.float32)

    return pl.pallas_call(
        body,
        grid=(G,),
        in_specs=[pl.BlockSpec((BR, 128), lambda i: (i, 0)),
                  pl.BlockSpec((BR, 128), lambda i: (i, 0)),
                  pl.BlockSpec((128, 128), lambda i: (0, 0)),
                  pl.BlockSpec((128, 128), lambda i: (0, 0)),
                  pl.BlockSpec((1, 128), lambda i: (0, 0)),
                  pl.BlockSpec((128, 128), lambda i: (0, 0)),
                  pl.BlockSpec((BR, 1), lambda i: (i, 0))],
        out_specs=pl.BlockSpec((BR, 128), lambda i: (i, 0)),
        out_shape=jax.ShapeDtypeStruct((N, 128), jnp.float32),
    )(x, posp, Wx, Wpp, b, W0, dinv)


def _tc_combine_stats(acc, y, dinv):
    def body(a_ref, y_ref, d_ref, o_ref, st_ref):
        i = pl.program_id(0)
        o = d_ref[...] * (a_ref[0] + a_ref[1] + y_ref[...])
        o_ref[...] = o
        s1 = jnp.sum(o, axis=0, keepdims=True)
        s2 = jnp.sum(o * o, axis=0, keepdims=True)
        part = jnp.concatenate(
            [s1, s2, jnp.zeros((6, 128), jnp.float32)], axis=0)

        @pl.when(i == 0)
        def _():
            st_ref[...] = part

        @pl.when(i > 0)
        def _():
            st_ref[...] += part

    return pl.pallas_call(
        body,
        grid=(G,),
        in_specs=[pl.BlockSpec((2, BR, 128), lambda i: (0, i, 0)),
                  pl.BlockSpec((BR, 128), lambda i: (i, 0)),
                  pl.BlockSpec((BR, 1), lambda i: (i, 0))],
        out_specs=[pl.BlockSpec((BR, 128), lambda i: (i, 0)),
                   pl.BlockSpec((8, 128), lambda i: (0, 0))],
        out_shape=[jax.ShapeDtypeStruct((N, 128), jnp.float32),
                   jax.ShapeDtypeStruct((8, 128), jnp.float32)],
    )(acc, y, dinv)


def _bn_block(o, st, g, b):
    mean = st[0:1, :] * (1.0 / N)
    ex2 = st[1:2, :] * (1.0 / N)
    var = ex2 - mean * mean
    rstd = lax.rsqrt(var + 1e-5)
    return jnp.maximum((o - mean) * rstd * g + b, 0.0)


def _tc_apply_matmul(out, st, g, b, Wn, dinv):
    def body(o_ref, st_ref, g_ref, b_ref, w_ref, d_ref, y_ref):
        h = _bn_block(o_ref[...], st_ref[...], g_ref[...], b_ref[...])
        y_ref[...] = d_ref[...] * jnp.dot(
            h, w_ref[...], preferred_element_type=jnp.float32)

    return pl.pallas_call(
        body,
        grid=(G,),
        in_specs=[pl.BlockSpec((BR, 128), lambda i: (i, 0)),
                  pl.BlockSpec((8, 128), lambda i: (0, 0)),
                  pl.BlockSpec((1, 128), lambda i: (0, 0)),
                  pl.BlockSpec((1, 128), lambda i: (0, 0)),
                  pl.BlockSpec((128, 128), lambda i: (0, 0)),
                  pl.BlockSpec((BR, 1), lambda i: (i, 0))],
        out_specs=pl.BlockSpec((BR, 128), lambda i: (i, 0)),
        out_shape=jax.ShapeDtypeStruct((N, 128), jnp.float32),
    )(out, st, g, b, Wn, dinv)


def _tc_apply_bn(out, st, g, b):
    def body(o_ref, st_ref, g_ref, b_ref, h_ref):
        h_ref[...] = _bn_block(o_ref[...], st_ref[...], g_ref[...], b_ref[...])

    return pl.pallas_call(
        body,
        grid=(G,),
        in_specs=[pl.BlockSpec((BR, 128), lambda i: (i, 0)),
                  pl.BlockSpec((8, 128), lambda i: (0, 0)),
                  pl.BlockSpec((1, 128), lambda i: (0, 0)),
                  pl.BlockSpec((1, 128), lambda i: (0, 0))],
        out_specs=pl.BlockSpec((BR, 128), lambda i: (i, 0)),
        out_shape=jax.ShapeDtypeStruct((N, 128), jnp.float32),
    )(out, st, g, b)


def _tc_predictor(pool_acc, cntT, W1, b1, W2, b2):
    def body(a_ref, c_ref, w1_ref, b1_ref, w2_ref, b2_ref, p_ref):
        cnt = jnp.sum(c_ref[...], axis=1, keepdims=True)
        cnt = jnp.maximum(cnt, 1.0)
        emb = (a_ref[0, :NG, :] + a_ref[1, :NG, :]) / cnt
        hid = jnp.maximum(
            jnp.dot(emb, w1_ref[...],
                    preferred_element_type=jnp.float32) + b1_ref[...], 0.0)
        p_ref[...] = jnp.dot(
            hid, w2_ref[...], preferred_element_type=jnp.float32) + b2_ref[...]

    return pl.pallas_call(
        body,
        grid=(1,),
        in_specs=[pl.BlockSpec((2, NPOOL, 128), lambda i: (0, 0, 0)),
                  pl.BlockSpec((NG, 32), lambda i: (0, 0)),
                  pl.BlockSpec((128, 128), lambda i: (0, 0)),
                  pl.BlockSpec((1, 128), lambda i: (0, 0)),
                  pl.BlockSpec((128, 19), lambda i: (0, 0)),
                  pl.BlockSpec((1, 19), lambda i: (0, 0))],
        out_specs=pl.BlockSpec((NG, 19), lambda i: (0, 0)),
        out_shape=jax.ShapeDtypeStruct((NG, 19), jnp.float32),
    )(pool_acc, cntT, W1, b1, W2, b2)


def kernel(x, pos, edge_index, batch, lin_W, lin_b, conv_W, conv_b, bn_g,
           bn_b, pred_W1, pred_b1, pred_W2, pred_b2):
    del conv_b
    src = edge_index[0].astype(jnp.int32)
    dst = edge_index[1].astype(jnp.int32)
    bat = batch.astype(jnp.int32)
    src_p = jnp.concatenate(
        [src, jnp.zeros((EPAD - E,), jnp.int32)]).reshape(-1, CH)
    dst_p = jnp.concatenate(
        [dst, jnp.full((EPAD - E,), N, jnp.int32)]).reshape(-1, CH)
    bat_pad = jnp.concatenate([bat, jnp.full((NP - N,), NG, jnp.int32)])
    psrc = jnp.concatenate(
        [jnp.arange(N, dtype=jnp.int32),
         jnp.zeros((EPOOL - N,), jnp.int32)]).reshape(-1, PCW)
    pdst = jnp.concatenate(
        [bat, jnp.full((EPOOL - N,), NG, jnp.int32)]).reshape(-1, PCW)
    posp = jnp.pad(pos, ((0, 0), (0, 125)))
    Wx = lin_W[:D]
    Wpp = jnp.pad(lin_W[D:D + 3], ((0, 125), (0, 0)))
    zeros_sc = jnp.zeros((NP // 16, 128), jnp.float32)

    deg_hist, cnt_hist = _sc_hists(dst, bat_pad)
    dinv = _tc_dinv(deg_hist.T[:N])
    y = _tc_proj_matmul(x, posp, Wx, Wpp, lin_b.reshape(1, 128),
                        conv_W[0], dinv)
    for i in range(4):
        acc = _sc_scatter_edges(y, src_p, dst_p, zeros_sc)
        out, st = _tc_combine_stats(acc, y, dinv)
        g = bn_g[i].reshape(1, 128)
        b = bn_b[i].reshape(1, 128)
        if i < 3:
            y = _tc_apply_matmul(out, st, g, b, conv_W[i + 1], dinv)
        else:
            h = _tc_apply_bn(out, st, g, b)
    pool = _sc_scatter_pool(h, psrc, pdst, zeros_sc)
    return _tc_predictor(pool, cnt_hist.T, pred_W1, pred_b1.reshape(1, 128),
                         pred_W2, pred_b2.reshape(1, 19))


# speedup vs baseline: 1.1693x; 1.1693x over previous
import functools

import jax
import jax.numpy as jnp
from jax import lax
from jax.experimental import pallas as pl
from jax.experimental.pallas import tpu as pltpu
from jax.experimental.pallas import tpu_sc as plsc

N = 10000
E = 320000
D = 128
NG = 256
NP = 10240
CH = 128
NCH0 = 80
NCH1 = 80
EPAD = 16 * (NCH0 + NCH1) * CH
NPOOL = 512
PCH = 8
PCW = 40
EPOOL = 32 * PCH * PCW
BR = 400
G = N // BR

_MESH = plsc.VectorSubcoreMesh(core_axis_name="c", subcore_axis_name="s")


def _make_sc_scatter(n_rows, n0, n1, ch):
    rp = n_rows // 16
    assert n0 == n1
    nch = n0
    n_steps = 2 if nch % 16 == 0 else 1
    cps = nch // n_steps
    assert cps % 2 == 0 and (n_steps == 1 or cps % 8 == 0)

    @functools.partial(
        pl.kernel,
        out_type=jax.ShapeDtypeStruct((2, n_rows, 128), jnp.float32),
        mesh=_MESH,
        scratch_types=[
            pltpu.VMEM((cps, ch), jnp.int32),
            pltpu.VMEM((cps, ch), jnp.int32),
            pltpu.VMEM((ch, 128), jnp.float32),
            pltpu.VMEM((ch, 128), jnp.float32),
            pltpu.VMEM_SHARED((n_rows, 128), jnp.float32),
            pltpu.SemaphoreType.DMA,
            pltpu.SemaphoreType.DMA,
        ],
        compiler_params=pltpu.CompilerParams(needs_layout_passes=False),
    )
    def k(y_hbm, src_hbm, dst_hbm, zeros_hbm, out_hbm, src_v, dst_v, r0, r1,
          acc_sh, sem0, sem1):
        c = lax.axis_index("c")
        s = lax.axis_index("s")
        wid = c * 16 + s
        pltpu.sync_copy(zeros_hbm.at[pl.ds(0, rp)], acc_sh.at[pl.ds(s * rp, rp)])

        def step(t, carry):
            base = wid * nch + t * cps
            pltpu.sync_copy(src_hbm.at[pl.ds(base, cps)], src_v)
            pltpu.sync_copy(dst_hbm.at[pl.ds(base, cps)], dst_v)
            pltpu.async_copy(y_hbm.at[src_v.at[0]], r0, sem0)

            def body(j, carry2):
                e = 2 * j
                pltpu.async_copy(y_hbm.at[src_v.at[e + 1]], r1, sem1)
                pltpu.make_async_copy(y_hbm.at[src_v.at[e]], r0, sem0).wait()
                pltpu.sync_copy(r0, acc_sh.at[dst_v.at[e]], add=True)

                @pl.when(j < cps // 2 - 1)
                def _():
                    pltpu.async_copy(y_hbm.at[src_v.at[e + 2]], r0, sem0)

                pltpu.make_async_copy(y_hbm.at[src_v.at[e + 1]], r1, sem1).wait()
                pltpu.sync_copy(r1, acc_sh.at[dst_v.at[e + 1]], add=True)
                return carry2

            lax.fori_loop(0, cps // 2, body, 0)
            return carry

        lax.fori_loop(0, n_steps, step, 0)
        plsc.subcore_barrier()
        pltpu.sync_copy(acc_sh.at[pl.ds(s * rp, rp)],
                        out_hbm.at[c].at[pl.ds(s * rp, rp)])

    return k


def _sc_hists(dst, bat_pad):
    pt_d = E // 32
    pt_b = NP // 32

    @functools.partial(
        pl.kernel,
        out_type=[jax.ShapeDtypeStruct((32, NP), jnp.float32),
                  jax.ShapeDtypeStruct((32, NPOOL), jnp.float32)],
        mesh=_MESH,
        scratch_types=[
            pltpu.VMEM((pt_d,), jnp.int32),
            pltpu.VMEM((pt_b,), jnp.int32),
            pltpu.VMEM((NP,), jnp.float32),
            pltpu.VMEM((NPOOL,), jnp.float32),
        ],
        compiler_params=pltpu.CompilerParams(needs_layout_passes=False),
    )
    def k(dst_hbm, bat_hbm, deg_hbm, cnt_hbm, dv, bv, dh, chh):
        c = lax.axis_index("c")
        s = lax.axis_index("s")
        wid = c * 16 + s
        pltpu.sync_copy(dst_hbm.at[pl.ds(wid * pt_d, pt_d)], dv)
        pltpu.sync_copy(bat_hbm.at[pl.ds(wid * pt_b, pt_b)], bv)
        zeros = jnp.zeros((16,), jnp.float32)
        ones = jnp.ones((16,), jnp.float32)

        def zd(j, carry):
            dh[pl.ds(j * 16, 16)] = zeros
            return carry

        lax.fori_loop(0, NP // 16, zd, 0)

        def zc(j, carry):
            chh[pl.ds(j * 16, 16)] = zeros
            return carry

        lax.fori_loop(0, NPOOL // 16, zc, 0)

        def bd(j, carry):
            plsc.addupdate_scatter(dh, [dv[pl.ds(j * 16, 16)]], ones)
            return carry

        lax.fori_loop(0, pt_d // 16, bd, 0)

        def bb(j, carry):
            plsc.addupdate_scatter(chh, [bv[pl.ds(j * 16, 16)]], ones)
            return carry

        lax.fori_loop(0, pt_b // 16, bb, 0)
        pltpu.sync_copy(dh, deg_hbm.at[wid])
        pltpu.sync_copy(chh, cnt_hbm.at[wid])

    return k(dst, bat_pad)


_sc_scatter_edges = _make_sc_scatter(NP, NCH0, NCH1, CH)
_sc_scatter_pool = _make_sc_scatter(NPOOL, PCH, PCH, PCW)


def _tc_dinv(histT):
    def body(h_ref, d_ref):
        deg = jnp.sum(h_ref[...], axis=1, keepdims=True) + 1.0
        d_ref[...] = lax.rsqrt(deg)

    return pl.pallas_call(
        body,
        grid=(G,),
        in_specs=[pl.BlockSpec((BR, 32), lambda i: (i, 0))],
        out_specs=pl.BlockSpec((BR, 1), lambda i: (i, 0)),
        out_shape=jax.ShapeDtypeStruct((N, 1), jnp.float32),
    )(histT)


def _tc_proj_matmul(x, posp, Wx, Wpp, b, W0, dinv):
    def body(x_ref, p_ref, wx_ref, wp_ref, b_ref, w0_ref, d_ref, y_ref):
        h = jnp.maximum(
            jnp.dot(x_ref[...], wx_ref[...], preferred_element_type=jnp.float32)
            + jnp.dot(p_ref[...], wp_ref[...], preferred_element_type=jnp.float32)
            + b_ref[...], 0.0)
        y_ref[...] = d_ref[...] * jnp.dot(
            h, w0_ref[...], preferred_element_type=jnp.float32)

    return pl.pallas_call(
        body,
        grid=(G,),
        in_specs=[pl.BlockSpec((BR, 128), lambda i: (i, 0)),
                  pl.BlockSpec((BR, 128), lambda i: (i, 0)),
                  pl.BlockSpec((128, 128), lambda i: (0, 0)),
                  pl.BlockSpec((128, 128), lambda i: (0, 0)),
                  pl.BlockSpec((1, 128), lambda i: (0, 0)),
                  pl.BlockSpec((128, 128), lambda i: (0, 0)),
                  pl.BlockSpec((BR, 1), lambda i: (i, 0))],
        out_specs=pl.BlockSpec((BR, 128), lambda i: (i, 0)),
        out_shape=jax.ShapeDtypeStruct((N, 128), jnp.float32),
    )(x, posp, Wx, Wpp, b, W0, dinv)


def _tc_combine_stats(acc, y, dinv):
    def body(a_ref, y_ref, d_ref, o_ref, st_ref):
        i = pl.program_id(0)
        o = d_ref[...] * (a_ref[0] + a_ref[1] + y_ref[...])
        o_ref[...] = o
        s1 = jnp.sum(o, axis=0, keepdims=True)
        s2 = jnp.sum(o * o, axis=0, keepdims=True)
        part = jnp.concatenate(
            [s1, s2, jnp.zeros((6, 128), jnp.float32)], axis=0)

        @pl.when(i == 0)
        def _():
            st_ref[...] = part

        @pl.when(i > 0)
        def _():
            st_ref[...] += part

    return pl.pallas_call(
        body,
        grid=(G,),
        in_specs=[pl.BlockSpec((2, BR, 128), lambda i: (0, i, 0)),
                  pl.BlockSpec((BR, 128), lambda i: (i, 0)),
                  pl.BlockSpec((BR, 1), lambda i: (i, 0))],
        out_specs=[pl.BlockSpec((BR, 128), lambda i: (i, 0)),
                   pl.BlockSpec((8, 128), lambda i: (0, 0))],
        out_shape=[jax.ShapeDtypeStruct((N, 128), jnp.float32),
                   jax.ShapeDtypeStruct((8, 128), jnp.float32)],
    )(acc, y, dinv)


def _bn_block(o, st, g, b):
    mean = st[0:1, :] * (1.0 / N)
    ex2 = st[1:2, :] * (1.0 / N)
    var = ex2 - mean * mean
    rstd = lax.rsqrt(var + 1e-5)
    return jnp.maximum((o - mean) * rstd * g + b, 0.0)


def _tc_apply_matmul(out, st, g, b, Wn, dinv):
    def body(o_ref, st_ref, g_ref, b_ref, w_ref, d_ref, y_ref):
        h = _bn_block(o_ref[...], st_ref[...], g_ref[...], b_ref[...])
        y_ref[...] = d_ref[...] * jnp.dot(
            h, w_ref[...], preferred_element_type=jnp.float32)

    return pl.pallas_call(
        body,
        grid=(G,),
        in_specs=[pl.BlockSpec((BR, 128), lambda i: (i, 0)),
                  pl.BlockSpec((8, 128), lambda i: (0, 0)),
                  pl.BlockSpec((1, 128), lambda i: (0, 0)),
                  pl.BlockSpec((1, 128), lambda i: (0, 0)),
                  pl.BlockSpec((128, 128), lambda i: (0, 0)),
                  pl.BlockSpec((BR, 1), lambda i: (i, 0))],
        out_specs=pl.BlockSpec((BR, 128), lambda i: (i, 0)),
        out_shape=jax.ShapeDtypeStruct((N, 128), jnp.float32),
    )(out, st, g, b, Wn, dinv)


def _tc_apply_bn(out, st, g, b):
    def body(o_ref, st_ref, g_ref, b_ref, h_ref):
        h_ref[...] = _bn_block(o_ref[...], st_ref[...], g_ref[...], b_ref[...])

    return pl.pallas_call(
        body,
        grid=(G,),
        in_specs=[pl.BlockSpec((BR, 128), lambda i: (i, 0)),
                  pl.BlockSpec((8, 128), lambda i: (0, 0)),
                  pl.BlockSpec((1, 128), lambda i: (0, 0)),
                  pl.BlockSpec((1, 128), lambda i: (0, 0))],
        out_specs=pl.BlockSpec((BR, 128), lambda i: (i, 0)),
        out_shape=jax.ShapeDtypeStruct((N, 128), jnp.float32),
    )(out, st, g, b)


def _tc_predictor(pool_acc, cntT, W1, b1, W2, b2):
    def body(a_ref, c_ref, w1_ref, b1_ref, w2_ref, b2_ref, p_ref):
        cnt = jnp.sum(c_ref[...], axis=1, keepdims=True)
        cnt = jnp.maximum(cnt, 1.0)
        emb = (a_ref[0, :NG, :] + a_ref[1, :NG, :]) / cnt
        hid = jnp.maximum(
            jnp.dot(emb, w1_ref[...],
                    preferred_element_type=jnp.float32) + b1_ref[...], 0.0)
        p_ref[...] = jnp.dot(
            hid, w2_ref[...], preferred_element_type=jnp.float32) + b2_ref[...]

    return pl.pallas_call(
        body,
        grid=(1,),
        in_specs=[pl.BlockSpec((2, NPOOL, 128), lambda i: (0, 0, 0)),
                  pl.BlockSpec((NG, 32), lambda i: (0, 0)),
                  pl.BlockSpec((128, 128), lambda i: (0, 0)),
                  pl.BlockSpec((1, 128), lambda i: (0, 0)),
                  pl.BlockSpec((128, 19), lambda i: (0, 0)),
                  pl.BlockSpec((1, 19), lambda i: (0, 0))],
        out_specs=pl.BlockSpec((NG, 19), lambda i: (0, 0)),
        out_shape=jax.ShapeDtypeStruct((NG, 19), jnp.float32),
    )(pool_acc, cntT, W1, b1, W2, b2)


def kernel(x, pos, edge_index, batch, lin_W, lin_b, conv_W, conv_b, bn_g,
           bn_b, pred_W1, pred_b1, pred_W2, pred_b2):
    del conv_b
    src = edge_index[0].astype(jnp.int32)
    dst = edge_index[1].astype(jnp.int32)
    bat = batch.astype(jnp.int32)
    src_p = jnp.concatenate(
        [src, jnp.zeros((EPAD - E,), jnp.int32)]).reshape(-1, CH)
    dst_p = jnp.concatenate(
        [dst, jnp.full((EPAD - E,), N, jnp.int32)]).reshape(-1, CH)
    bat_pad = jnp.concatenate([bat, jnp.full((NP - N,), NG, jnp.int32)])
    psrc = jnp.concatenate(
        [jnp.arange(N, dtype=jnp.int32),
         jnp.zeros((EPOOL - N,), jnp.int32)]).reshape(-1, PCW)
    pdst = jnp.concatenate(
        [bat, jnp.full((EPOOL - N,), NG, jnp.int32)]).reshape(-1, PCW)
    posp = jnp.pad(pos, ((0, 0), (0, 125)))
    Wx = lin_W[:D]
    Wpp = jnp.pad(lin_W[D:D + 3], ((0, 125), (0, 0)))
    zeros_sc = jnp.zeros((NP // 16, 128), jnp.float32)

    deg_hist, cnt_hist = _sc_hists(dst, bat_pad)
    dinv = _tc_dinv(deg_hist.T[:N])
    y = _tc_proj_matmul(x, posp, Wx, Wpp, lin_b.reshape(1, 128),
                        conv_W[0], dinv)
    for i in range(4):
        acc = _sc_scatter_edges(y, src_p, dst_p, zeros_sc)
        out, st = _tc_combine_stats(acc, y, dinv)
        g = bn_g[i].reshape(1, 128)
        b = bn_b[i].reshape(1, 128)
        if i < 3:
            y = _tc_apply_matmul(out, st, g, b, conv_W[i + 1], dinv)
        else:
            h = _tc_apply_bn(out, st, g, b)
    pool = _sc_scatter_pool(h, psrc, pdst, zeros_sc)
    return _tc_predictor(pool, cnt_hist.T, pred_W1, pred_b1.reshape(1, 128),
                         pred_W2, pred_b2.reshape(1, 19))


# speedup vs baseline: 1.9997x; 1.7102x over previous
import functools

import jax
import jax.numpy as jnp
from jax import lax
from jax.experimental import pallas as pl
from jax.experimental.pallas import tpu as pltpu
from jax.experimental.pallas import tpu_sc as plsc

N = 10000
E = 320000
D = 128
NG = 256
NP = 10240
CH = 128
HALF = 5120
SQ = 3072
QCH = SQ // CH
ACC_R = 5376
ACC_RP = ACC_R // 16
YR_T = HALF // 16
PH_CH = 2 * QCH
NPOOL = 512
PCH = 8
PCW = 40
EPOOL = 32 * PCH * PCW
BR = 400
G = N // BR

_MESH = plsc.VectorSubcoreMesh(core_axis_name="c", subcore_axis_name="s")


def _sc_partition(src, dst):
    pt = E // 32

    @functools.partial(
        pl.kernel,
        out_type=[jax.ShapeDtypeStruct((4, 32 * SQ), jnp.int32),
                  jax.ShapeDtypeStruct((4, 32 * SQ), jnp.int32)],
        mesh=_MESH,
        scratch_types=[pltpu.VMEM((pt,), jnp.int32),
                       pltpu.VMEM((pt,), jnp.int32)] +
                      [pltpu.VMEM((SQ + 16,), jnp.int32) for _ in range(8)],
        compiler_params=pltpu.CompilerParams(needs_layout_passes=False),
    )
    def k(src_hbm, dst_hbm, srcq_hbm, dstq_hbm, sv, dv,
          s0, d0, s1, d1, s2, d2, s3, d3):
        c = lax.axis_index("c")
        s = lax.axis_index("s")
        wid = c * 16 + s
        pltpu.sync_copy(src_hbm.at[pl.ds(wid * pt, pt)], sv)
        pltpu.sync_copy(dst_hbm.at[pl.ds(wid * pt, pt)], dv)
        iota = lax.iota(jnp.int32, 16)
        sbufs = (s0, s1, s2, s3)
        dbufs = (d0, d1, d2, d3)

        def body(j, offs):
            sc = sv[pl.ds(j * 16, 16)]
            dc = dv[pl.ds(j * 16, 16)]
            s_lo = sc < HALF
            d_lo = dc < HALF
            sl = jnp.where(s_lo, sc, sc - HALF)
            dl = jnp.where(d_lo, dc, dc - HALF)
            masks = (s_lo & d_lo, s_lo & (~d_lo),
                     (~s_lo) & d_lo, (~s_lo) & (~d_lo))
            new = []
            for q in range(4):
                o = offs[q]
                plsc.store_compressed(sbufs[q].at[pl.ds(o, 16)], sl,
                                      mask=masks[q])
                plsc.store_compressed(dbufs[q].at[pl.ds(o, 16)], dl,
                                      mask=masks[q])
                new.append(o + jnp.sum(masks[q].astype(jnp.int32)))
            return tuple(new)

        zero = jnp.int32(0)
        offs = lax.fori_loop(0, pt // 16, body, (zero, zero, zero, zero))

        for q in range(4):
            o = offs[q]
            nfill = (SQ - o + 15) // 16

            def fb(kk, carry, q=q, o=o):
                pos = o + kk * 16
                sbufs[q][pl.ds(pos, 16)] = jnp.zeros((16,), jnp.int32)
                dbufs[q][pl.ds(pos, 16)] = HALF + ((pos + iota) % 128)
                return carry

            lax.fori_loop(0, nfill, fb, 0)
            pltpu.sync_copy(sbufs[q].at[pl.ds(0, SQ)],
                            srcq_hbm.at[q].at[pl.ds(wid * SQ, SQ)])
            pltpu.sync_copy(dbufs[q].at[pl.ds(0, SQ)],
                            dstq_hbm.at[q].at[pl.ds(wid * SQ, SQ)])

    return k(src, dst)


def _sc_scatter_edges(yh, srcq, dstq, zeros_sc):
    @functools.partial(
        pl.kernel,
        out_type=jax.ShapeDtypeStruct((2, ACC_R, 128), jnp.float32),
        mesh=_MESH,
        scratch_types=[
            pltpu.VMEM((PH_CH, CH), jnp.int32),
            pltpu.VMEM((PH_CH, CH), jnp.int32),
            pltpu.VMEM((CH, 128), jnp.float32),
            pltpu.VMEM_SHARED((ACC_R, 128), jnp.float32),
            pltpu.VMEM_SHARED((HALF, 128), jnp.float32),
            pltpu.SemaphoreType.DMA,
        ],
        compiler_params=pltpu.CompilerParams(needs_layout_passes=False),
    )
    def k(yh_hbm, srcq_hbm, dstq_hbm, zeros_hbm, out_hbm, src_v, dst_v,
          rows_v, acc_sh, y_sh, sem):
        c = lax.axis_index("c")
        s = lax.axis_index("s")
        pltpu.sync_copy(zeros_hbm.at[pl.ds(0, ACC_RP)],
                        acc_sh.at[pl.ds(s * ACC_RP, ACC_RP)])
        for p in range(2):
            pltpu.sync_copy(yh_hbm.at[p].at[pl.ds(s * YR_T, YR_T)],
                            y_sh.at[pl.ds(s * YR_T, YR_T)])
            plsc.subcore_barrier()
            q = 2 * p + c
            pltpu.sync_copy(srcq_hbm.at[q].at[pl.ds(s * PH_CH, PH_CH)], src_v)
            pltpu.sync_copy(dstq_hbm.at[q].at[pl.ds(s * PH_CH, PH_CH)], dst_v)

            def body(j, carry):
                pltpu.async_copy(y_sh.at[src_v.at[j]], rows_v, sem).wait()
                pltpu.sync_copy(rows_v, acc_sh.at[dst_v.at[j]], add=True)
                return carry

            lax.fori_loop(0, PH_CH, body, 0)
            plsc.subcore_barrier()
        pltpu.sync_copy(acc_sh.at[pl.ds(s * ACC_RP, ACC_RP)],
                        out_hbm.at[c].at[pl.ds(s * ACC_RP, ACC_RP)])

    return k(yh, srcq, dstq, zeros_sc)


def _make_sc_scatter(n_rows, n0, n1, ch):
    rp = n_rows // 16
    assert n0 == n1
    nch = n0

    @functools.partial(
        pl.kernel,
        out_type=jax.ShapeDtypeStruct((2, n_rows, 128), jnp.float32),
        mesh=_MESH,
        scratch_types=[
            pltpu.VMEM((nch, ch), jnp.int32),
            pltpu.VMEM((nch, ch), jnp.int32),
            pltpu.VMEM((ch, 128), jnp.float32),
            pltpu.VMEM_SHARED((n_rows, 128), jnp.float32),
            pltpu.SemaphoreType.DMA,
        ],
        compiler_params=pltpu.CompilerParams(needs_layout_passes=False),
    )
    def k(y_hbm, src_hbm, dst_hbm, zeros_hbm, out_hbm, src_v, dst_v, rows_v,
          acc_sh, sem):
        c = lax.axis_index("c")
        s = lax.axis_index("s")
        wid = c * 16 + s
        pltpu.sync_copy(zeros_hbm.at[pl.ds(0, rp)], acc_sh.at[pl.ds(s * rp, rp)])
        pltpu.sync_copy(src_hbm.at[pl.ds(wid * nch, nch)], src_v)
        pltpu.sync_copy(dst_hbm.at[pl.ds(wid * nch, nch)], dst_v)

        def body(j, carry):
            pltpu.async_copy(y_hbm.at[src_v.at[j]], rows_v, sem).wait()
            pltpu.sync_copy(rows_v, acc_sh.at[dst_v.at[j]], add=True)
            return carry

        lax.fori_loop(0, nch, body, 0)
        plsc.subcore_barrier()
        pltpu.sync_copy(acc_sh.at[pl.ds(s * rp, rp)],
                        out_hbm.at[c].at[pl.ds(s * rp, rp)])

    return k


def _sc_hists(dst, bat_pad):
    pt_d = E // 32
    pt_b = NP // 32

    @functools.partial(
        pl.kernel,
        out_type=[jax.ShapeDtypeStruct((32, NP), jnp.float32),
                  jax.ShapeDtypeStruct((32, NPOOL), jnp.float32)],
        mesh=_MESH,
        scratch_types=[
            pltpu.VMEM((pt_d,), jnp.int32),
            pltpu.VMEM((pt_b,), jnp.int32),
            pltpu.VMEM((NP,), jnp.float32),
            pltpu.VMEM((NPOOL,), jnp.float32),
        ],
        compiler_params=pltpu.CompilerParams(needs_layout_passes=False),
    )
    def k(dst_hbm, bat_hbm, deg_hbm, cnt_hbm, dv, bv, dh, chh):
        c = lax.axis_index("c")
        s = lax.axis_index("s")
        wid = c * 16 + s
        pltpu.sync_copy(dst_hbm.at[pl.ds(wid * pt_d, pt_d)], dv)
        pltpu.sync_copy(bat_hbm.at[pl.ds(wid * pt_b, pt_b)], bv)
        zeros = jnp.zeros((16,), jnp.float32)
        ones = jnp.ones((16,), jnp.float32)

        def zd(j, carry):
            dh[pl.ds(j * 16, 16)] = zeros
            return carry

        lax.fori_loop(0, NP // 16, zd, 0)

        def zc(j, carry):
            chh[pl.ds(j * 16, 16)] = zeros
            return carry

        lax.fori_loop(0, NPOOL // 16, zc, 0)

        def bd(j, carry):
            plsc.addupdate_scatter(dh, [dv[pl.ds(j * 16, 16)]], ones)
            return carry

        lax.fori_loop(0, pt_d // 16, bd, 0)

        def bb(j, carry):
            plsc.addupdate_scatter(chh, [bv[pl.ds(j * 16, 16)]], ones)
            return carry

        lax.fori_loop(0, pt_b // 16, bb, 0)
        pltpu.sync_copy(dh, deg_hbm.at[wid])
        pltpu.sync_copy(chh, cnt_hbm.at[wid])

    return k(dst, bat_pad)


_sc_scatter_pool = _make_sc_scatter(NPOOL, PCH, PCH, PCW)


def _tc_dinv(histT):
    def body(h_ref, d_ref):
        deg = jnp.sum(h_ref[...], axis=1, keepdims=True) + 1.0
        d_ref[...] = lax.rsqrt(deg)

    return pl.pallas_call(
        body,
        grid=(G,),
        in_specs=[pl.BlockSpec((BR, 32), lambda i: (i, 0))],
        out_specs=pl.BlockSpec((BR, 1), lambda i: (i, 0)),
        out_shape=jax.ShapeDtypeStruct((N, 1), jnp.float32),
    )(histT)


def _tc_proj_matmul(x, posp, Wx, Wpp, b, W0, dinv):
    def body(x_ref, p_ref, wx_ref, wp_ref, b_ref, w0_ref, d_ref, y_ref):
        h = jnp.maximum(
            jnp.dot(x_ref[...], wx_ref[...], preferred_element_type=jnp.float32)
            + jnp.dot(p_ref[...], wp_ref[...], preferred_element_type=jnp.float32)
            + b_ref[...], 0.0)
        y_ref[...] = d_ref[...] * jnp.dot(
            h, w0_ref[...], preferred_element_type=jnp.float32)

    return pl.pallas_call(
        body,
        grid=(G,),
        in_specs=[pl.BlockSpec((BR, 128), lambda i: (i, 0)),
                  pl.BlockSpec((BR, 128), lambda i: (i, 0)),
                  pl.BlockSpec((128, 128), lambda i: (0, 0)),
                  pl.BlockSpec((128, 128), lambda i: (0, 0)),
                  pl.BlockSpec((1, 128), lambda i: (0, 0)),
                  pl.BlockSpec((128, 128), lambda i: (0, 0)),
                  pl.BlockSpec((BR, 1), lambda i: (i, 0))],
        out_specs=pl.BlockSpec((BR, 128), lambda i: (i, 0)),
        out_shape=jax.ShapeDtypeStruct((N, 128), jnp.float32),
    )(x, posp, Wx, Wpp, b, W0, dinv)


def _tc_combine_stats(acc, y, dinv):
    def body(a_ref, y_ref, d_ref, o_ref, st_ref):
        i = pl.program_id(0)
        o = d_ref[...] * (a_ref[...] + y_ref[...])
        o_ref[...] = o
        s1 = jnp.sum(o, axis=0, keepdims=True)
        s2 = jnp.sum(o * o, axis=0, keepdims=True)
        part = jnp.concatenate(
            [s1, s2, jnp.zeros((6, 128), jnp.float32)], axis=0)

        @pl.when(i == 0)
        def _():
            st_ref[...] = part

        @pl.when(i > 0)
        def _():
            st_ref[...] += part

    return pl.pallas_call(
        body,
        grid=(G,),
        in_specs=[pl.BlockSpec((BR, 128), lambda i: (i, 0)),
                  pl.BlockSpec((BR, 128), lambda i: (i, 0)),
                  pl.BlockSpec((BR, 1), lambda i: (i, 0))],
        out_specs=[pl.BlockSpec((BR, 128), lambda i: (i, 0)),
                   pl.BlockSpec((8, 128), lambda i: (0, 0))],
        out_shape=[jax.ShapeDtypeStruct((N, 128), jnp.float32),
                   jax.ShapeDtypeStruct((8, 128), jnp.float32)],
    )(acc, y, dinv)


def _bn_block(o, st, g, b):
    mean = st[0:1, :] * (1.0 / N)
    ex2 = st[1:2, :] * (1.0 / N)
    var = ex2 - mean * mean
    rstd = lax.rsqrt(var + 1e-5)
    return jnp.maximum((o - mean) * rstd * g + b, 0.0)


def _tc_apply_matmul(out, st, g, b, Wn, dinv):
    def body(o_ref, st_ref, g_ref, b_ref, w_ref, d_ref, y_ref):
        h = _bn_block(o_ref[...], st_ref[...], g_ref[...], b_ref[...])
        y_ref[...] = d_ref[...] * jnp.dot(
            h, w_ref[...], preferred_element_type=jnp.float32)

    return pl.pallas_call(
        body,
        grid=(G,),
        in_specs=[pl.BlockSpec((BR, 128), lambda i: (i, 0)),
                  pl.BlockSpec((8, 128), lambda i: (0, 0)),
                  pl.BlockSpec((1, 128), lambda i: (0, 0)),
                  pl.BlockSpec((1, 128), lambda i: (0, 0)),
                  pl.BlockSpec((128, 128), lambda i: (0, 0)),
                  pl.BlockSpec((BR, 1), lambda i: (i, 0))],
        out_specs=pl.BlockSpec((BR, 128), lambda i: (i, 0)),
        out_shape=jax.ShapeDtypeStruct((N, 128), jnp.float32),
    )(out, st, g, b, Wn, dinv)


def _tc_apply_bn(out, st, g, b):
    def body(o_ref, st_ref, g_ref, b_ref, h_ref):
        h_ref[...] = _bn_block(o_ref[...], st_ref[...], g_ref[...], b_ref[...])

    return pl.pallas_call(
        body,
        grid=(G,),
        in_specs=[pl.BlockSpec((BR, 128), lambda i: (i, 0)),
                  pl.BlockSpec((8, 128), lambda i: (0, 0)),
                  pl.BlockSpec((1, 128), lambda i: (0, 0)),
                  pl.BlockSpec((1, 128), lambda i: (0, 0))],
        out_specs=pl.BlockSpec((BR, 128), lambda i: (i, 0)),
        out_shape=jax.ShapeDtypeStruct((N, 128), jnp.float32),
    )(out, st, g, b)


def _tc_predictor(pool_acc, cntT, W1, b1, W2, b2):
    def body(a_ref, c_ref, w1_ref, b1_ref, w2_ref, b2_ref, p_ref):
        cnt = jnp.sum(c_ref[...], axis=1, keepdims=True)
        cnt = jnp.maximum(cnt, 1.0)
        emb = (a_ref[0, :NG, :] + a_ref[1, :NG, :]) / cnt
        hid = jnp.maximum(
            jnp.dot(emb, w1_ref[...],
                    preferred_element_type=jnp.float32) + b1_ref[...], 0.0)
        p_ref[...] = jnp.dot(
            hid, w2_ref[...], preferred_element_type=jnp.float32) + b2_ref[...]

    return pl.pallas_call(
        body,
        grid=(1,),
        in_specs=[pl.BlockSpec((2, NPOOL, 128), lambda i: (0, 0, 0)),
                  pl.BlockSpec((NG, 32), lambda i: (0, 0)),
                  pl.BlockSpec((128, 128), lambda i: (0, 0)),
                  pl.BlockSpec((1, 128), lambda i: (0, 0)),
                  pl.BlockSpec((128, 19), lambda i: (0, 0)),
                  pl.BlockSpec((1, 19), lambda i: (0, 0))],
        out_specs=pl.BlockSpec((NG, 19), lambda i: (0, 0)),
        out_shape=jax.ShapeDtypeStruct((NG, 19), jnp.float32),
    )(pool_acc, cntT, W1, b1, W2, b2)


def kernel(x, pos, edge_index, batch, lin_W, lin_b, conv_W, conv_b, bn_g,
           bn_b, pred_W1, pred_b1, pred_W2, pred_b2):
    del conv_b
    src = edge_index[0].astype(jnp.int32)
    dst = edge_index[1].astype(jnp.int32)
    bat = batch.astype(jnp.int32)
    bat_pad = jnp.concatenate([bat, jnp.full((NP - N,), NG, jnp.int32)])
    psrc = jnp.concatenate(
        [jnp.arange(N, dtype=jnp.int32),
         jnp.zeros((EPOOL - N,), jnp.int32)]).reshape(-1, PCW)
    pdst = jnp.concatenate(
        [bat, jnp.full((EPOOL - N,), NG, jnp.int32)]).reshape(-1, PCW)
    posp = jnp.pad(pos, ((0, 0), (0, 125)))
    Wx = lin_W[:D]
    Wpp = jnp.pad(lin_W[D:D + 3], ((0, 125), (0, 0)))
    zeros_sc = jnp.zeros((NP // 16, 128), jnp.float32)

    deg_hist, cnt_hist = _sc_hists(dst, bat_pad)
    srcq, dstq = _sc_partition(src, dst)
    srcq3 = srcq.reshape(4, 32 * QCH, CH)
    dstq3 = dstq.reshape(4, 32 * QCH, CH)
    dinv = _tc_dinv(deg_hist.T[:N])
    y = _tc_proj_matmul(x, posp, Wx, Wpp, lin_b.reshape(1, 128),
                        conv_W[0], dinv)
    for i in range(4):
        yh = jnp.pad(y, ((0, 2 * HALF - N), (0, 0))).reshape(2, HALF, 128)
        acc2 = _sc_scatter_edges(yh, srcq3, dstq3, zeros_sc)
        acc = jnp.concatenate([acc2[0, :HALF], acc2[1, :N - HALF]], axis=0)
        out, st = _tc_combine_stats(acc, y, dinv)
        g = bn_g[i].reshape(1, 128)
        b = bn_b[i].reshape(1, 128)
        if i < 3:
            y = _tc_apply_matmul(out, st, g, b, conv_W[i + 1], dinv)
        else:
            h = _tc_apply_bn(out, st, g, b)
    pool = _sc_scatter_pool(h, psrc, pdst, zeros_sc)
    return _tc_predictor(pool, cnt_hist.T, pred_W1, pred_b1.reshape(1, 128),
                         pred_W2, pred_b2.reshape(1, 19))


# speedup vs baseline: 2.4861x; 1.2432x over previous
import functools

import jax
import jax.numpy as jnp
from jax import lax
from jax.experimental import pallas as pl
from jax.experimental.pallas import tpu as pltpu
from jax.experimental.pallas import tpu_sc as plsc

N = 10000
E = 320000
D = 128
NG = 256
NP = 10240
CH = 128
HALF = 5120
SQ = 3072
QCH = SQ // CH
ACC_R = 5376
ACC_RP = ACC_R // 16
YR_T = HALF // 16
PH_CH = 2 * QCH
NPOOL = 512
PCH = 8
PCW = 40
EPOOL = 32 * PCH * PCW
BR = 400
G = N // BR

_MESH = plsc.VectorSubcoreMesh(core_axis_name="c", subcore_axis_name="s")


def _sc_partition(src, dst):
    pt = E // 32

    @functools.partial(
        pl.kernel,
        out_type=[jax.ShapeDtypeStruct((4, 32 * SQ), jnp.int32),
                  jax.ShapeDtypeStruct((4, 32 * SQ), jnp.int32)],
        mesh=_MESH,
        scratch_types=[pltpu.VMEM((pt,), jnp.int32),
                       pltpu.VMEM((pt,), jnp.int32)] +
                      [pltpu.VMEM((SQ + 16,), jnp.int32) for _ in range(8)],
        compiler_params=pltpu.CompilerParams(needs_layout_passes=False),
    )
    def k(src_hbm, dst_hbm, srcq_hbm, dstq_hbm, sv, dv,
          s0, d0, s1, d1, s2, d2, s3, d3):
        c = lax.axis_index("c")
        s = lax.axis_index("s")
        wid = c * 16 + s
        pltpu.sync_copy(src_hbm.at[pl.ds(wid * pt, pt)], sv)
        pltpu.sync_copy(dst_hbm.at[pl.ds(wid * pt, pt)], dv)
        iota = lax.iota(jnp.int32, 16)
        sbufs = (s0, s1, s2, s3)
        dbufs = (d0, d1, d2, d3)

        def body(j, offs):
            sc = sv[pl.ds(j * 16, 16)]
            dc = dv[pl.ds(j * 16, 16)]
            s_lo = sc < HALF
            d_lo = dc < HALF
            sl = jnp.where(s_lo, sc, sc - HALF)
            dl = jnp.where(d_lo, dc, dc - HALF)
            masks = (s_lo & d_lo, s_lo & (~d_lo),
                     (~s_lo) & d_lo, (~s_lo) & (~d_lo))
            new = []
            for q in range(4):
                o = offs[q]
                plsc.store_compressed(sbufs[q].at[pl.ds(o, 16)], sl,
                                      mask=masks[q])
                plsc.store_compressed(dbufs[q].at[pl.ds(o, 16)], dl,
                                      mask=masks[q])
                new.append(o + jnp.sum(masks[q].astype(jnp.int32)))
            return tuple(new)

        zero = jnp.int32(0)
        offs = lax.fori_loop(0, pt // 16, body, (zero, zero, zero, zero))

        for q in range(4):
            o = offs[q]
            nfill = (SQ - o + 15) // 16

            def fb(kk, carry, q=q, o=o):
                pos = o + kk * 16
                sbufs[q][pl.ds(pos, 16)] = jnp.zeros((16,), jnp.int32)
                dbufs[q][pl.ds(pos, 16)] = HALF + ((pos + iota) % 128)
                return carry

            lax.fori_loop(0, nfill, fb, 0)
            pltpu.sync_copy(sbufs[q].at[pl.ds(0, SQ)],
                            srcq_hbm.at[q].at[pl.ds(wid * SQ, SQ)])
            pltpu.sync_copy(dbufs[q].at[pl.ds(0, SQ)],
                            dstq_hbm.at[q].at[pl.ds(wid * SQ, SQ)])

    return k(src, dst)


def _sc_scatter_edges(yh, srcq, dstq, zeros_sc):
    @functools.partial(
        pl.kernel,
        out_type=jax.ShapeDtypeStruct((2, ACC_R, 128), jnp.float32),
        mesh=_MESH,
        scratch_types=[
            pltpu.VMEM((PH_CH, CH), jnp.int32),
            pltpu.VMEM((PH_CH, CH), jnp.int32),
            pltpu.VMEM((CH, 128), jnp.float32),
            pltpu.VMEM((CH, 128), jnp.float32),
            pltpu.VMEM_SHARED((ACC_R, 128), jnp.float32),
            pltpu.VMEM_SHARED((HALF, 128), jnp.float32),
            pltpu.SemaphoreType.DMA,
            pltpu.SemaphoreType.DMA,
        ],
        compiler_params=pltpu.CompilerParams(needs_layout_passes=False),
    )
    def k(yh_hbm, srcq_hbm, dstq_hbm, zeros_hbm, out_hbm, src_v, dst_v,
          r0, r1, acc_sh, y_sh, sem0, sem1):
        c = lax.axis_index("c")
        s = lax.axis_index("s")
        pltpu.sync_copy(zeros_hbm.at[pl.ds(0, ACC_RP)],
                        acc_sh.at[pl.ds(s * ACC_RP, ACC_RP)])
        for p in range(2):
            pltpu.sync_copy(yh_hbm.at[p].at[pl.ds(s * YR_T, YR_T)],
                            y_sh.at[pl.ds(s * YR_T, YR_T)])
            plsc.subcore_barrier()
            q = 2 * p + c
            pltpu.sync_copy(srcq_hbm.at[q].at[pl.ds(s * PH_CH, PH_CH)], src_v)
            pltpu.sync_copy(dstq_hbm.at[q].at[pl.ds(s * PH_CH, PH_CH)], dst_v)
            pltpu.async_copy(y_sh.at[src_v.at[0]], r0, sem0)

            def body(j, carry):
                e = 2 * j
                pltpu.async_copy(y_sh.at[src_v.at[e + 1]], r1, sem1)
                pltpu.make_async_copy(y_sh.at[src_v.at[e]], r0, sem0).wait()
                pltpu.sync_copy(r0, acc_sh.at[dst_v.at[e]], add=True)

                @pl.when(j < PH_CH // 2 - 1)
                def _():
                    pltpu.async_copy(y_sh.at[src_v.at[e + 2]], r0, sem0)

                pltpu.make_async_copy(y_sh.at[src_v.at[e + 1]], r1, sem1).wait()
                pltpu.sync_copy(r1, acc_sh.at[dst_v.at[e + 1]], add=True)
                return carry

            lax.fori_loop(0, PH_CH // 2, body, 0)
            plsc.subcore_barrier()
        pltpu.sync_copy(acc_sh.at[pl.ds(s * ACC_RP, ACC_RP)],
                        out_hbm.at[c].at[pl.ds(s * ACC_RP, ACC_RP)])

    return k(yh, srcq, dstq, zeros_sc)


def _make_sc_scatter(n_rows, n0, n1, ch):
    rp = n_rows // 16
    assert n0 == n1
    nch = n0

    @functools.partial(
        pl.kernel,
        out_type=jax.ShapeDtypeStruct((2, n_rows, 128), jnp.float32),
        mesh=_MESH,
        scratch_types=[
            pltpu.VMEM((nch, ch), jnp.int32),
            pltpu.VMEM((nch, ch), jnp.int32),
            pltpu.VMEM((ch, 128), jnp.float32),
            pltpu.VMEM_SHARED((n_rows, 128), jnp.float32),
            pltpu.SemaphoreType.DMA,
        ],
        compiler_params=pltpu.CompilerParams(needs_layout_passes=False),
    )
    def k(y_hbm, src_hbm, dst_hbm, zeros_hbm, out_hbm, src_v, dst_v, rows_v,
          acc_sh, sem):
        c = lax.axis_index("c")
        s = lax.axis_index("s")
        wid = c * 16 + s
        pltpu.sync_copy(zeros_hbm.at[pl.ds(0, rp)], acc_sh.at[pl.ds(s * rp, rp)])
        pltpu.sync_copy(src_hbm.at[pl.ds(wid * nch, nch)], src_v)
        pltpu.sync_copy(dst_hbm.at[pl.ds(wid * nch, nch)], dst_v)

        def body(j, carry):
            pltpu.async_copy(y_hbm.at[src_v.at[j]], rows_v, sem).wait()
            pltpu.sync_copy(rows_v, acc_sh.at[dst_v.at[j]], add=True)
            return carry

        lax.fori_loop(0, nch, body, 0)
        plsc.subcore_barrier()
        pltpu.sync_copy(acc_sh.at[pl.ds(s * rp, rp)],
                        out_hbm.at[c].at[pl.ds(s * rp, rp)])

    return k


def _sc_hists(dst, bat_pad):
    pt_d = E // 32
    pt_b = NP // 32

    @functools.partial(
        pl.kernel,
        out_type=[jax.ShapeDtypeStruct((32, NP), jnp.float32),
                  jax.ShapeDtypeStruct((32, NPOOL), jnp.float32)],
        mesh=_MESH,
        scratch_types=[
            pltpu.VMEM((pt_d,), jnp.int32),
            pltpu.VMEM((pt_b,), jnp.int32),
            pltpu.VMEM((NP,), jnp.float32),
            pltpu.VMEM((NPOOL,), jnp.float32),
        ],
        compiler_params=pltpu.CompilerParams(needs_layout_passes=False),
    )
    def k(dst_hbm, bat_hbm, deg_hbm, cnt_hbm, dv, bv, dh, chh):
        c = lax.axis_index("c")
        s = lax.axis_index("s")
        wid = c * 16 + s
        pltpu.sync_copy(dst_hbm.at[pl.ds(wid * pt_d, pt_d)], dv)
        pltpu.sync_copy(bat_hbm.at[pl.ds(wid * pt_b, pt_b)], bv)
        zeros = jnp.zeros((16,), jnp.float32)
        ones = jnp.ones((16,), jnp.float32)

        def zd(j, carry):
            dh[pl.ds(j * 16, 16)] = zeros
            return carry

        lax.fori_loop(0, NP // 16, zd, 0)

        def zc(j, carry):
            chh[pl.ds(j * 16, 16)] = zeros
            return carry

        lax.fori_loop(0, NPOOL // 16, zc, 0)

        def bd(j, carry):
            plsc.addupdate_scatter(dh, [dv[pl.ds(j * 16, 16)]], ones)
            return carry

        lax.fori_loop(0, pt_d // 16, bd, 0)

        def bb(j, carry):
            plsc.addupdate_scatter(chh, [bv[pl.ds(j * 16, 16)]], ones)
            return carry

        lax.fori_loop(0, pt_b // 16, bb, 0)
        pltpu.sync_copy(dh, deg_hbm.at[wid])
        pltpu.sync_copy(chh, cnt_hbm.at[wid])

    return k(dst, bat_pad)


_sc_scatter_pool = _make_sc_scatter(NPOOL, PCH, PCH, PCW)


def _tc_dinv(histT):
    def body(h_ref, d_ref):
        deg = jnp.sum(h_ref[...], axis=1, keepdims=True) + 1.0
        d_ref[...] = lax.rsqrt(deg)

    return pl.pallas_call(
        body,
        grid=(G,),
        in_specs=[pl.BlockSpec((BR, 32), lambda i: (i, 0))],
        out_specs=pl.BlockSpec((BR, 1), lambda i: (i, 0)),
        out_shape=jax.ShapeDtypeStruct((N, 1), jnp.float32),
    )(histT)


def _tc_proj_matmul(x, posp, Wx, Wpp, b, W0, dinv):
    def body(x_ref, p_ref, wx_ref, wp_ref, b_ref, w0_ref, d_ref, y_ref):
        h = jnp.maximum(
            jnp.dot(x_ref[...], wx_ref[...], preferred_element_type=jnp.float32)
            + jnp.dot(p_ref[...], wp_ref[...], preferred_element_type=jnp.float32)
            + b_ref[...], 0.0)
        y_ref[...] = d_ref[...] * jnp.dot(
            h, w0_ref[...], preferred_element_type=jnp.float32)

    return pl.pallas_call(
        body,
        grid=(G,),
        in_specs=[pl.BlockSpec((BR, 128), lambda i: (i, 0)),
                  pl.BlockSpec((BR, 128), lambda i: (i, 0)),
                  pl.BlockSpec((128, 128), lambda i: (0, 0)),
                  pl.BlockSpec((128, 128), lambda i: (0, 0)),
                  pl.BlockSpec((1, 128), lambda i: (0, 0)),
                  pl.BlockSpec((128, 128), lambda i: (0, 0)),
                  pl.BlockSpec((BR, 1), lambda i: (i, 0))],
        out_specs=pl.BlockSpec((BR, 128), lambda i: (i, 0)),
        out_shape=jax.ShapeDtypeStruct((N, 128), jnp.float32),
    )(x, posp, Wx, Wpp, b, W0, dinv)


def _tc_combine_stats(acc, y, dinv):
    def body(a_ref, y_ref, d_ref, o_ref, st_ref):
        i = pl.program_id(0)
        o = d_ref[...] * (a_ref[...] + y_ref[...])
        o_ref[...] = o
        s1 = jnp.sum(o, axis=0, keepdims=True)
        s2 = jnp.sum(o * o, axis=0, keepdims=True)
        part = jnp.concatenate(
            [s1, s2, jnp.zeros((6, 128), jnp.float32)], axis=0)

        @pl.when(i == 0)
        def _():
            st_ref[...] = part

        @pl.when(i > 0)
        def _():
            st_ref[...] += part

    return pl.pallas_call(
        body,
        grid=(G,),
        in_specs=[pl.BlockSpec((BR, 128), lambda i: (i, 0)),
                  pl.BlockSpec((BR, 128), lambda i: (i, 0)),
                  pl.BlockSpec((BR, 1), lambda i: (i, 0))],
        out_specs=[pl.BlockSpec((BR, 128), lambda i: (i, 0)),
                   pl.BlockSpec((8, 128), lambda i: (0, 0))],
        out_shape=[jax.ShapeDtypeStruct((N, 128), jnp.float32),
                   jax.ShapeDtypeStruct((8, 128), jnp.float32)],
    )(acc, y, dinv)


def _bn_block(o, st, g, b):
    mean = st[0:1, :] * (1.0 / N)
    ex2 = st[1:2, :] * (1.0 / N)
    var = ex2 - mean * mean
    rstd = lax.rsqrt(var + 1e-5)
    return jnp.maximum((o - mean) * rstd * g + b, 0.0)


def _tc_apply_matmul(out, st, g, b, Wn, dinv):
    def body(o_ref, st_ref, g_ref, b_ref, w_ref, d_ref, y_ref):
        h = _bn_block(o_ref[...], st_ref[...], g_ref[...], b_ref[...])
        y_ref[...] = d_ref[...] * jnp.dot(
            h, w_ref[...], preferred_element_type=jnp.float32)

    return pl.pallas_call(
        body,
        grid=(G,),
        in_specs=[pl.BlockSpec((BR, 128), lambda i: (i, 0)),
                  pl.BlockSpec((8, 128), lambda i: (0, 0)),
                  pl.BlockSpec((1, 128), lambda i: (0, 0)),
                  pl.BlockSpec((1, 128), lambda i: (0, 0)),
                  pl.BlockSpec((128, 128), lambda i: (0, 0)),
                  pl.BlockSpec((BR, 1), lambda i: (i, 0))],
        out_specs=pl.BlockSpec((BR, 128), lambda i: (i, 0)),
        out_shape=jax.ShapeDtypeStruct((N, 128), jnp.float32),
    )(out, st, g, b, Wn, dinv)


def _tc_apply_bn(out, st, g, b):
    def body(o_ref, st_ref, g_ref, b_ref, h_ref):
        h_ref[...] = _bn_block(o_ref[...], st_ref[...], g_ref[...], b_ref[...])

    return pl.pallas_call(
        body,
        grid=(G,),
        in_specs=[pl.BlockSpec((BR, 128), lambda i: (i, 0)),
                  pl.BlockSpec((8, 128), lambda i: (0, 0)),
                  pl.BlockSpec((1, 128), lambda i: (0, 0)),
                  pl.BlockSpec((1, 128), lambda i: (0, 0))],
        out_specs=pl.BlockSpec((BR, 128), lambda i: (i, 0)),
        out_shape=jax.ShapeDtypeStruct((N, 128), jnp.float32),
    )(out, st, g, b)


def _tc_predictor(pool_acc, cntT, W1, b1, W2, b2):
    def body(a_ref, c_ref, w1_ref, b1_ref, w2_ref, b2_ref, p_ref):
        cnt = jnp.sum(c_ref[...], axis=1, keepdims=True)
        cnt = jnp.maximum(cnt, 1.0)
        emb = (a_ref[0, :NG, :] + a_ref[1, :NG, :]) / cnt
        hid = jnp.maximum(
            jnp.dot(emb, w1_ref[...],
                    preferred_element_type=jnp.float32) + b1_ref[...], 0.0)
        p_ref[...] = jnp.dot(
            hid, w2_ref[...], preferred_element_type=jnp.float32) + b2_ref[...]

    return pl.pallas_call(
        body,
        grid=(1,),
        in_specs=[pl.BlockSpec((2, NPOOL, 128), lambda i: (0, 0, 0)),
                  pl.BlockSpec((NG, 32), lambda i: (0, 0)),
                  pl.BlockSpec((128, 128), lambda i: (0, 0)),
                  pl.BlockSpec((1, 128), lambda i: (0, 0)),
                  pl.BlockSpec((128, 19), lambda i: (0, 0)),
                  pl.BlockSpec((1, 19), lambda i: (0, 0))],
        out_specs=pl.BlockSpec((NG, 19), lambda i: (0, 0)),
        out_shape=jax.ShapeDtypeStruct((NG, 19), jnp.float32),
    )(pool_acc, cntT, W1, b1, W2, b2)


def kernel(x, pos, edge_index, batch, lin_W, lin_b, conv_W, conv_b, bn_g,
           bn_b, pred_W1, pred_b1, pred_W2, pred_b2):
    del conv_b
    src = edge_index[0].astype(jnp.int32)
    dst = edge_index[1].astype(jnp.int32)
    bat = batch.astype(jnp.int32)
    bat_pad = jnp.concatenate([bat, jnp.full((NP - N,), NG, jnp.int32)])
    psrc = jnp.concatenate(
        [jnp.arange(N, dtype=jnp.int32),
         jnp.zeros((EPOOL - N,), jnp.int32)]).reshape(-1, PCW)
    pdst = jnp.concatenate(
        [bat, jnp.full((EPOOL - N,), NG, jnp.int32)]).reshape(-1, PCW)
    posp = jnp.pad(pos, ((0, 0), (0, 125)))
    Wx = lin_W[:D]
    Wpp = jnp.pad(lin_W[D:D + 3], ((0, 125), (0, 0)))
    zeros_sc = jnp.zeros((NP // 16, 128), jnp.float32)

    deg_hist, cnt_hist = _sc_hists(dst, bat_pad)
    srcq, dstq = _sc_partition(src, dst)
    srcq3 = srcq.reshape(4, 32 * QCH, CH)
    dstq3 = dstq.reshape(4, 32 * QCH, CH)
    dinv = _tc_dinv(deg_hist.T[:N])
    y = _tc_proj_matmul(x, posp, Wx, Wpp, lin_b.reshape(1, 128),
                        conv_W[0], dinv)
    for i in range(4):
        yh = jnp.pad(y, ((0, 2 * HALF - N), (0, 0))).reshape(2, HALF, 128)
        acc2 = _sc_scatter_edges(yh, srcq3, dstq3, zeros_sc)
        acc = jnp.concatenate([acc2[0, :HALF], acc2[1, :N - HALF]], axis=0)
        out, st = _tc_combine_stats(acc, y, dinv)
        g = bn_g[i].reshape(1, 128)
        b = bn_b[i].reshape(1, 128)
        if i < 3:
            y = _tc_apply_matmul(out, st, g, b, conv_W[i + 1], dinv)
        else:
            h = _tc_apply_bn(out, st, g, b)
    pool = _sc_scatter_pool(h, psrc, pdst, zeros_sc)
    return _tc_predictor(pool, cnt_hist.T, pred_W1, pred_b1.reshape(1, 128),
                         pred_W2, pred_b2.reshape(1, 19))
